# baseline (speedup 1.0000x reference)
import jax
import jax.numpy as jnp
from jax import lax
from jax.experimental import pallas as pl
from jax.experimental.pallas import tpu as pltpu
from jax.experimental.pallas import tpu_sc as plsc

K = 64
ROWS = 128
N = 32768
L = 16
NV = N // L
NWORKERS = 32
RPW = ROWS // NWORKERS
CAP = 2048
NEG_INF_KEY = -(2 ** 31)
POS_INF_IDX = 2 ** 31 - 1


def _scalar(v16):
    return jnp.squeeze(lax.slice(v16, (0,), (1,)))


def _key(v):
    u = lax.bitcast_convert_type(v, jnp.int32)
    return u ^ (lax.shift_right_arithmetic(u, 31) & 0x7FFFFFFF)


def _topk_body(x_hbm, o_hbm, rowbuf0, rowbuf1, rowbuf2,
               hist_v, ckey_v, cidx_v, tie_v, wstage_val, wstage_idx,
               wval_v, wgidx_v, sem_in0, sem_in1, sem_in2, sem_s):
    wid = lax.axis_index("s") * 2 + lax.axis_index("c")
    iota = lax.iota(jnp.int32, L)
    ones = jnp.ones((L,), jnp.int32)

    sems_in = [sem_in0, sem_in1, sem_in2]
    rowbufs = [rowbuf0, rowbuf1, rowbuf2]
    NBUF = 3

    row0 = wid * RPW
    for r in range(min(NBUF, RPW)):
        pltpu.async_copy(
            x_hbm.at[pl.ds((row0 + r) * N, N)], rowbufs[r], sems_in[r])

    for r in range(RPW):
        row = row0 + r
        base = row * N
        row_v = rowbufs[r % NBUF]

        pltpu.make_async_copy(
            x_hbm.at[pl.ds(base, N)], row_v, sems_in[r % NBUF]).wait()

        @plsc.parallel_loop(0, 4096 // L, unroll=8)
        def _(j):
            hist_v[pl.ds(j * L, L)] = jnp.zeros((L,), jnp.int32)

        @plsc.parallel_loop(0, NV, unroll=8)
        def _(i):
            key = _key(row_v[pl.ds(i * L, L)])
            b = lax.shift_right_arithmetic(key, 20) + 2048
            plsc.addupdate_scatter(hist_v, [b], ones)

        def scan_cond(c):
            return jnp.logical_not(c[2])

        def scan_body(c):
            j, cum, found, b1 = c
            hv = hist_v[pl.ds(j * L, L)]
            s = jnp.sum(hv)
            found_here = (cum + s) >= K
            pref = plsc.cumsum(hv)
            suf_in = s - pref + hv
            cross = (cum + suf_in) >= K
            npos = jnp.sum(cross.astype(jnp.int32))
            b1_here = j * L + npos - 1
            b1_n = jnp.where(found_here, b1_here, b1)
            cum_n = jnp.where(found_here, cum, cum + s)
            return (j - 1, cum_n, found | found_here, b1_n)

        init = (jnp.int32(4096 // L - 1), jnp.int32(0), False, jnp.int32(0))
        _, _, _, b1 = lax.while_loop(scan_cond, scan_body, init)

        e = lax.shift_left(b1 - 2048, 20)

        @plsc.parallel_loop(0, NV, unroll=4, carry=jnp.int32(0))
        def coff_final(i, coff):
            v = row_v[pl.ds(i * L, L)]
            kv = _key(v)
            in_b = kv >= e
            plsc.store_compressed(ckey_v.at[pl.ds(coff, L)], kv, mask=in_b)
            plsc.store_compressed(
                cidx_v.at[pl.ds(coff, L)], iota + i * L, mask=in_b)
            cnt = _scalar(plsc.all_reduce_population_count(in_b))
            return jnp.minimum(coff + cnt, CAP - L)

        c = coff_final

        if r + NBUF < RPW:
            pltpu.async_copy(
                x_hbm.at[pl.ds((row + NBUF) * N, N)],
                rowbufs[r % NBUF],
                sems_in[r % NBUF],
            )

        ckey_v[pl.ds(c, L)] = jnp.full((L,), NEG_INF_KEY, jnp.int32)
        nv = (c + L - 1) // L

        def bs_body(it, t):
            cand = t + lax.shift_left(1, 30 - it)

            @plsc.parallel_loop(0, nv, unroll=4,
                                carry=jnp.zeros((L,), jnp.int32))
            def acc_final(j, acc):
                kv = ckey_v[pl.ds(j * L, L)]
                return acc + plsc.all_reduce_population_count(kv >= cand)

            cnt = _scalar(acc_final)
            return jnp.where(cnt >= K, cand, t)

        t = lax.fori_loop(11, 31, bs_body, e)

        @plsc.parallel_loop(0, nv, unroll=4, carry=jnp.zeros((L,), jnp.int32))
        def gt_final(j, acc):
            kv = ckey_v[pl.ds(j * L, L)]
            return acc + plsc.all_reduce_population_count(kv > t)

        cnt_gt = _scalar(gt_final)
        ties_needed = K - cnt_gt

        @plsc.parallel_loop(0, nv, unroll=4, carry=jnp.int32(0))
        def nt_final(j, toff):
            kv = ckey_v[pl.ds(j * L, L)]
            iv = cidx_v[pl.ds(j * L, L)]
            m = kv == t
            plsc.store_compressed(tie_v.at[pl.ds(toff, L)], iv, mask=m)
            cnt = _scalar(plsc.all_reduce_population_count(m))
            return jnp.minimum(toff + cnt, CAP - L)

        ntie = nt_final
        tie_v[pl.ds(ntie, L)] = jnp.full((L,), POS_INF_IDX, jnp.int32)
        nvt = (ntie + L - 1) // L

        def ts_body(it, v):
            cand = v + lax.shift_left(1, 14 - it)

            @plsc.parallel_loop(0, nvt, unroll=2,
                                carry=jnp.zeros((L,), jnp.int32))
            def tacc_final(j, acc):
                tv = tie_v[pl.ds(j * L, L)]
                return acc + plsc.all_reduce_population_count(tv < cand)

            cnt = _scalar(tacc_final)
            return jnp.where(cnt < ties_needed, cand, v)

        vstar = lax.fori_loop(0, 15, ts_body, jnp.int32(0))

        if r > 0:
            pltpu.make_async_copy(
                wval_v, o_hbm.at[wgidx_v], sem_s).wait()

        @plsc.parallel_loop(0, nv, unroll=2, carry=jnp.int32(0))
        def woff_final(j, woff):
            kv = ckey_v[pl.ds(j * L, L)]
            iv = cidx_v[pl.ds(j * L, L)]
            m = (kv > t) | ((kv == t) & (iv <= vstar))
            wv = lax.bitcast_convert_type(
                jnp.maximum(kv, 0), jnp.float32)
            plsc.store_compressed(
                wstage_val.at[pl.ds(woff, L)], wv, mask=m)
            plsc.store_compressed(
                wstage_idx.at[pl.ds(woff, L)], iv + base, mask=m)
            cnt = _scalar(plsc.all_reduce_population_count(m))
            return woff + cnt

        for j in range(K // L):
            wval_v[pl.ds(j * L, L)] = wstage_val[pl.ds(j * L, L)]
            wgidx_v[pl.ds(j * L, L)] = wstage_idx[pl.ds(j * L, L)]

        pltpu.async_copy(wval_v, o_hbm.at[wgidx_v], sem_s)

    pltpu.make_async_copy(wval_v, o_hbm.at[wgidx_v], sem_s).wait()


@jax.jit
def _topk_sc(x_flat):
    mesh = plsc.VectorSubcoreMesh(core_axis_name="c", subcore_axis_name="s")
    f = pl.kernel(
        _topk_body,
        out_type=(),
        mesh=mesh,
        scratch_types=[
            pltpu.VMEM((N,), jnp.float32),
            pltpu.VMEM((N,), jnp.float32),
            pltpu.VMEM((N,), jnp.float32),
            pltpu.VMEM((4096,), jnp.int32),
            pltpu.VMEM((CAP,), jnp.int32),
            pltpu.VMEM((CAP,), jnp.int32),
            pltpu.VMEM((CAP,), jnp.int32),
            pltpu.VMEM((K + L,), jnp.float32),
            pltpu.VMEM((K + L,), jnp.int32),
            pltpu.VMEM((K,), jnp.float32),
            pltpu.VMEM((K,), jnp.int32),
            pltpu.SemaphoreType.DMA,
            pltpu.SemaphoreType.DMA,
            pltpu.SemaphoreType.DMA,
            pltpu.SemaphoreType.DMA,
        ],
        compiler_params=pltpu.CompilerParams(needs_layout_passes=False),
    )
    o_ref = jax.new_ref(jnp.zeros((ROWS * N,), jnp.float32))
    f(x_flat, o_ref)
    return jax.freeze(o_ref)


def kernel(x):
    out = _topk_sc(x.reshape(-1))
    return out.reshape(ROWS, N)


# speedup vs baseline: 1.0259x; 1.0259x over previous
import jax
import jax.numpy as jnp
from jax import lax
from jax.experimental import pallas as pl
from jax.experimental.pallas import tpu as pltpu
from jax.experimental.pallas import tpu_sc as plsc

K = 64
ROWS = 128
N = 32768
L = 16
NV = N // L
NB = NV // L
NG = NB * L
NWORKERS = 32
RPW = ROWS // NWORKERS
CAP = 2048
NEG_INF_KEY = -(2 ** 31)
POS_INF_IDX = 2 ** 31 - 1


def _scalar(v16):
    return jnp.squeeze(lax.slice(v16, (0,), (1,)))


def _key(v):
    u = lax.bitcast_convert_type(v, jnp.int32)
    return u ^ (lax.shift_right_arithmetic(u, 31) & 0x7FFFFFFF)


def _topk_body(x_hbm, o_hbm, rowbuf0, rowbuf1, rowbuf2, gmax_v,
               hist_v, ckey_v, cidx_v, tie_v, wstage_val, wstage_idx,
               wval_v, wgidx_v, sem_in0, sem_in1, sem_in2, sem_s):
    wid = lax.axis_index("s") * 2 + lax.axis_index("c")
    iota = lax.iota(jnp.int32, L)
    ones = jnp.ones((L,), jnp.int32)

    sems_in = [sem_in0, sem_in1, sem_in2]
    rowbufs = [rowbuf0, rowbuf1, rowbuf2]
    NBUF = 3

    row0 = wid * RPW
    for r in range(min(NBUF, RPW)):
        pltpu.async_copy(
            x_hbm.at[pl.ds((row0 + r) * N, N)], rowbufs[r], sems_in[r])

    for r in range(RPW):
        row = row0 + r
        base = row * N
        row_v = rowbufs[r % NBUF]

        pltpu.make_async_copy(
            x_hbm.at[pl.ds(base, N)], row_v, sems_in[r % NBUF]).wait()

        @plsc.parallel_loop(0, NB, unroll=1, carry=jnp.int32(0))
        def gdone(b, g):
            acc = row_v[pl.ds(b * 256, L)]
            for i in range(1, L):
                acc = jnp.maximum(acc, row_v[pl.ds(b * 256 + i * L, L)])
            gmax_v[pl.ds(b * L, L)] = acc
            return g | _scalar(lax.bitcast_convert_type(acc, jnp.int32))

        guard0 = jnp.minimum(lax.shift_right_logical(gdone, 16),
                             jnp.int32(0))

        @plsc.parallel_loop(0, 4096 // L, unroll=8)
        def _(j):
            hist_v[pl.ds(j * L, L)] = jnp.zeros((L,), jnp.int32)

        @plsc.parallel_loop(0, NG // L, unroll=8)
        def _(i):
            key = _key(gmax_v[pl.ds(i * L + guard0, L)])
            b = lax.shift_right_arithmetic(key, 20) + 2048
            plsc.addupdate_scatter(hist_v, [b], ones)

        def scan_cond(c):
            return jnp.logical_not(c[2])

        def scan_body(c):
            j, cum, found, b1 = c
            hv = hist_v[pl.ds(j * L, L)]
            s = jnp.sum(hv)
            found_here = (cum + s) >= K
            pref = plsc.cumsum(hv)
            suf_in = s - pref + hv
            cross = (cum + suf_in) >= K
            npos = jnp.sum(cross.astype(jnp.int32))
            b1_here = j * L + npos - 1
            b1_n = jnp.where(found_here, b1_here, b1)
            cum_n = jnp.where(found_here, cum, cum + s)
            return (j - 1, cum_n, found | found_here, b1_n)

        init = (jnp.int32(4096 // L - 1), jnp.int32(0), False, jnp.int32(0))
        _, _, _, b1 = lax.while_loop(scan_cond, scan_body, init)

        e = lax.shift_left(b1 - 2048, 20)

        @plsc.parallel_loop(0, NV, unroll=4, carry=jnp.int32(0))
        def coff_final(i, coff):
            v = row_v[pl.ds(i * L, L)]
            kv = _key(v)
            in_b = kv >= e
            plsc.store_compressed(ckey_v.at[pl.ds(coff, L)], kv, mask=in_b)
            plsc.store_compressed(
                cidx_v.at[pl.ds(coff, L)], iota + i * L, mask=in_b)
            cnt = _scalar(plsc.all_reduce_population_count(in_b))
            return jnp.minimum(coff + cnt, CAP - L)

        c = coff_final

        if r + NBUF < RPW:
            pltpu.async_copy(
                x_hbm.at[pl.ds((row + NBUF) * N, N)],
                rowbufs[r % NBUF],
                sems_in[r % NBUF],
            )

        ckey_v[pl.ds(c, L)] = jnp.full((L,), NEG_INF_KEY, jnp.int32)
        nv = (c + L - 1) // L

        def bs_body(it, t):
            cand = t + lax.shift_left(1, 30 - it)

            @plsc.parallel_loop(0, nv, unroll=4,
                                carry=jnp.zeros((L,), jnp.int32))
            def acc_final(j, acc):
                kv = ckey_v[pl.ds(j * L, L)]
                return acc + plsc.all_reduce_population_count(kv >= cand)

            cnt = _scalar(acc_final)
            return jnp.where((cand > t) & (cnt >= K), cand, t)

        t = lax.fori_loop(0, 31, bs_body, e)

        @plsc.parallel_loop(0, nv, unroll=4, carry=jnp.zeros((L,), jnp.int32))
        def gt_final(j, acc):
            kv = ckey_v[pl.ds(j * L, L)]
            return acc + plsc.all_reduce_population_count(kv > t)

        cnt_gt = _scalar(gt_final)
        ties_needed = K - cnt_gt

        @plsc.parallel_loop(0, nv, unroll=4, carry=jnp.int32(0))
        def nt_final(j, toff):
            kv = ckey_v[pl.ds(j * L, L)]
            iv = cidx_v[pl.ds(j * L, L)]
            m = kv == t
            plsc.store_compressed(tie_v.at[pl.ds(toff, L)], iv, mask=m)
            cnt = _scalar(plsc.all_reduce_population_count(m))
            return jnp.minimum(toff + cnt, CAP - L)

        ntie = nt_final
        tie_v[pl.ds(ntie, L)] = jnp.full((L,), POS_INF_IDX, jnp.int32)
        nvt = (ntie + L - 1) // L

        def ts_body(it, v):
            cand = v + lax.shift_left(1, 14 - it)

            @plsc.parallel_loop(0, nvt, unroll=2,
                                carry=jnp.zeros((L,), jnp.int32))
            def tacc_final(j, acc):
                tv = tie_v[pl.ds(j * L, L)]
                return acc + plsc.all_reduce_population_count(tv < cand)

            cnt = _scalar(tacc_final)
            return jnp.where(cnt < ties_needed, cand, v)

        vstar = lax.fori_loop(0, 15, ts_body, jnp.int32(0))

        if r > 0:
            pltpu.make_async_copy(
                wval_v, o_hbm.at[wgidx_v], sem_s).wait()

        @plsc.parallel_loop(0, nv, unroll=2, carry=jnp.int32(0))
        def woff_final(j, woff):
            kv = ckey_v[pl.ds(j * L, L)]
            iv = cidx_v[pl.ds(j * L, L)]
            m = (kv > t) | ((kv == t) & (iv <= vstar))
            wv = lax.bitcast_convert_type(
                jnp.maximum(kv, 0), jnp.float32)
            plsc.store_compressed(
                wstage_val.at[pl.ds(woff, L)], wv, mask=m)
            plsc.store_compressed(
                wstage_idx.at[pl.ds(woff, L)], iv + base, mask=m)
            cnt = _scalar(plsc.all_reduce_population_count(m))
            return woff + cnt

        for j in range(K // L):
            wval_v[pl.ds(j * L, L)] = wstage_val[pl.ds(j * L, L)]
            wgidx_v[pl.ds(j * L, L)] = wstage_idx[pl.ds(j * L, L)]

        pltpu.async_copy(wval_v, o_hbm.at[wgidx_v], sem_s)

    pltpu.make_async_copy(wval_v, o_hbm.at[wgidx_v], sem_s).wait()


@jax.jit
def _topk_sc(x_flat):
    mesh = plsc.VectorSubcoreMesh(core_axis_name="c", subcore_axis_name="s")
    f = pl.kernel(
        _topk_body,
        out_type=(),
        mesh=mesh,
        scratch_types=[
            pltpu.VMEM((N,), jnp.float32),
            pltpu.VMEM((N,), jnp.float32),
            pltpu.VMEM((N,), jnp.float32),
            pltpu.VMEM((NG,), jnp.float32),
            pltpu.VMEM((4096,), jnp.int32),
            pltpu.VMEM((CAP,), jnp.int32),
            pltpu.VMEM((CAP,), jnp.int32),
            pltpu.VMEM((CAP,), jnp.int32),
            pltpu.VMEM((K + L,), jnp.float32),
            pltpu.VMEM((K + L,), jnp.int32),
            pltpu.VMEM((K,), jnp.float32),
            pltpu.VMEM((K,), jnp.int32),
            pltpu.SemaphoreType.DMA,
            pltpu.SemaphoreType.DMA,
            pltpu.SemaphoreType.DMA,
            pltpu.SemaphoreType.DMA,
        ],
        compiler_params=pltpu.CompilerParams(needs_layout_passes=False),
    )
    o_ref = jax.new_ref(jnp.zeros((ROWS * N,), jnp.float32))
    f(x_flat, o_ref)
    return jax.freeze(o_ref)


def kernel(x):
    out = _topk_sc(x.reshape(-1))
    return out.reshape(ROWS, N)
